# Initial kernel scaffold; baseline (speedup 1.0000x reference)
import jax
import jax.numpy as jnp
from jax.experimental import pallas as pl


def kernel(x, edge_index, conv_W0, conv_b0, conv_W1, conv_b1, gru_Wih0, gru_Whh0, gru_bih0, gru_bhh0, gru_Wih1, gru_Whh1, gru_bih1, gru_bhh1, lin_W, lin_b):
    raise NotImplementedError("write your pallas kernel here")


# speedup vs baseline: 8.6767x; 8.6767x over previous
import functools

import jax
import jax.numpy as jnp
from jax import lax
from jax.experimental import pallas as pl
from jax.experimental.pallas import tpu as pltpu
from jax.experimental.pallas import tpu_sc as plsc

N = 10000
E = 320000
H = 128

NPAD = 10240
NCORES = 2
NSUB = 16
NTILES = NCORES * NSUB
CHUNK = 128
EPT = 10240
NCHUNK = EPT // CHUNK
EPAD = NTILES * EPT
RPT = NPAD // NSUB
RB = 1024

_mesh = plsc.VectorSubcoreMesh(core_axis_name="c", subcore_axis_name="s")


@functools.partial(
    pl.kernel,
    out_type=jax.ShapeDtypeStruct((NCORES, NPAD, 16), jnp.float32),
    mesh=_mesh,
    scratch_types=[
        pltpu.VMEM((NCHUNK, CHUNK), jnp.int32),
        pltpu.VMEM((CHUNK, 16), jnp.float32),
        pltpu.VMEM((CHUNK, 16), jnp.float32),
        pltpu.VMEM_SHARED((NPAD, 16), jnp.float32),
    ],
)
def _deg_kernel(dsti_hbm, out_hbm, dst_v, zbuf, obuf, acc):
    c = lax.axis_index("c")
    s = lax.axis_index("s")
    t = c * NSUB + s
    pltpu.sync_copy(dsti_hbm.at[t], dst_v)

    def _fill(i, _):
        zbuf[i] = jnp.zeros((16,), jnp.float32)
        obuf[i] = jnp.ones((16,), jnp.float32)
        return 0

    lax.fori_loop(0, CHUNK, _fill, 0)

    base = s * RPT
    for k in range(RPT // CHUNK):
        pltpu.sync_copy(zbuf, acc.at[pl.ds(base + k * CHUNK, CHUNK), :])
    plsc.subcore_barrier()

    def _body(j, _):
        pltpu.sync_copy(obuf, acc.at[dst_v.at[j]], add=True)
        return 0

    lax.fori_loop(0, NCHUNK, _body, 0)
    plsc.subcore_barrier()

    for k in range(RPT // CHUNK):
        pltpu.sync_copy(acc.at[pl.ds(base + k * CHUNK, CHUNK), :], zbuf)
        pltpu.sync_copy(zbuf, out_hbm.at[c, pl.ds(base + k * CHUNK, CHUNK), :])


@functools.partial(
    pl.kernel,
    out_type=jax.ShapeDtypeStruct((NCORES, NPAD, H), jnp.float32),
    mesh=_mesh,
    scratch_types=[
        pltpu.VMEM((NCHUNK, CHUNK), jnp.int32),
        pltpu.VMEM((NCHUNK, CHUNK), jnp.int32),
        pltpu.VMEM((CHUNK, H), jnp.float32),
        pltpu.VMEM_SHARED((NPAD, H), jnp.float32),
        pltpu.SemaphoreType.DMA,
    ],
)
def _agg_kernel(table_hbm, srci_hbm, dsti_hbm, out_hbm, src_v, dst_v, rows, acc, sem):
    c = lax.axis_index("c")
    s = lax.axis_index("s")
    t = c * NSUB + s
    pltpu.sync_copy(srci_hbm.at[t], src_v)
    pltpu.sync_copy(dsti_hbm.at[t], dst_v)

    def _zero(i, _):
        for cc in range(H // 16):
            rows[i, pl.ds(cc * 16, 16)] = jnp.zeros((16,), jnp.float32)
        return 0

    lax.fori_loop(0, CHUNK, _zero, 0)

    base = s * RPT
    for k in range(RPT // CHUNK):
        pltpu.sync_copy(rows, acc.at[pl.ds(base + k * CHUNK, CHUNK), :])
    plsc.subcore_barrier()

    def _body(j, _):
        pltpu.async_copy(table_hbm.at[src_v.at[j]], rows, sem).wait()
        pltpu.sync_copy(rows, acc.at[dst_v.at[j]], add=True)
        return 0

    lax.fori_loop(0, NCHUNK, _body, 0)
    plsc.subcore_barrier()

    for k in range(RPT // CHUNK):
        pltpu.sync_copy(acc.at[pl.ds(base + k * CHUNK, CHUNK), :], rows)
        pltpu.sync_copy(rows, out_hbm.at[c, pl.ds(base + k * CHUNK, CHUNK), :])


def _prep_body(cnt0, cnt1, x, w0, o_scaled, o_dinv):
    deg = cnt0[:, 0:1] + cnt1[:, 0:1] + 1.0
    dinv = lax.rsqrt(deg)
    hw = jnp.dot(x[:], w0[:], preferred_element_type=jnp.float32)
    o_scaled[:] = hw * dinv
    o_dinv[:] = jnp.broadcast_to(dinv, (RB, H))


def _gru(gx, bhh):
    r = jax.nn.sigmoid(gx[:, 0:H] + bhh[:, 0:H])
    z = jax.nn.sigmoid(gx[:, H:2 * H] + bhh[:, H:2 * H])
    n = jnp.tanh(gx[:, 2 * H:3 * H] + r * bhh[:, 2 * H:3 * H])
    return (1.0 - z) * n


def _mid_body(agg0, agg1, scaled, dinv, b, wihT, bih, bhh, w_next, o_scaled2):
    conv = dinv[:] * (agg0[:] + agg1[:] + scaled[:]) + b[:]
    a = jnp.maximum(conv, 0.0)
    gx = jnp.dot(a, wihT[:], preferred_element_type=jnp.float32) + bih[:]
    h1 = _gru(gx, bhh[:])
    hw2 = jnp.dot(h1, w_next[:], preferred_element_type=jnp.float32)
    o_scaled2[:] = hw2 * dinv[:]


def _fin_body(agg0, agg1, scaled, dinv, b, wihT, bih, bhh, linWT, linb, o):
    conv = dinv[:] * (agg0[:] + agg1[:] + scaled[:]) + b[:]
    a = jnp.maximum(conv, 0.0)
    gx = jnp.dot(a, wihT[:], preferred_element_type=jnp.float32) + bih[:]
    h2 = _gru(gx, bhh[:])
    o[:] = jnp.dot(h2, linWT[:], preferred_element_type=jnp.float32) + linb[:]


_row = pl.BlockSpec((RB, H), lambda i: (i, 0))
_row16 = pl.BlockSpec((RB, 16), lambda i: (i, 0))
_w128 = pl.BlockSpec((H, H), lambda i: (0, 0))
_w384 = pl.BlockSpec((H, 3 * H), lambda i: (0, 0))
_b128 = pl.BlockSpec((1, H), lambda i: (0, 0))
_b384 = pl.BlockSpec((1, 3 * H), lambda i: (0, 0))
_GRID = (NPAD // RB,)

_prep_call = pl.pallas_call(
    _prep_body,
    grid=_GRID,
    in_specs=[_row16, _row16, _row, _w128],
    out_specs=[_row, _row],
    out_shape=[
        jax.ShapeDtypeStruct((NPAD, H), jnp.float32),
        jax.ShapeDtypeStruct((NPAD, H), jnp.float32),
    ],
)

_mid_call = pl.pallas_call(
    _mid_body,
    grid=_GRID,
    in_specs=[_row, _row, _row, _row, _b128, _w384, _b384, _b384, _w128],
    out_specs=[_row],
    out_shape=[jax.ShapeDtypeStruct((NPAD, H), jnp.float32)],
)

_fin_call = pl.pallas_call(
    _fin_body,
    grid=_GRID,
    in_specs=[_row, _row, _row, _row, _b128, _w384, _b384, _b384, _w128, _b128],
    out_specs=[_row],
    out_shape=[jax.ShapeDtypeStruct((NPAD, H), jnp.float32)],
)


@jax.jit
def kernel(x, edge_index, conv_W0, conv_b0, conv_W1, conv_b1, gru_Wih0,
           gru_Whh0, gru_bih0, gru_bhh0, gru_Wih1, gru_Whh1, gru_bih1,
           gru_bhh1, lin_W, lin_b):
    x_pad = jnp.zeros((NPAD, H), jnp.float32).at[:N].set(x)
    src = jnp.concatenate(
        [edge_index[0], jnp.zeros((EPAD - E,), jnp.int32)]).reshape(
            NTILES, NCHUNK, CHUNK)
    dst = jnp.concatenate(
        [edge_index[1], jnp.full((EPAD - E,), N, jnp.int32)]).reshape(
            NTILES, NCHUNK, CHUNK)
    b0 = conv_b0.reshape(1, H)
    b1 = conv_b1.reshape(1, H)
    wih0T = gru_Wih0.T
    wih1T = gru_Wih1.T
    bih0 = gru_bih0.reshape(1, 3 * H)
    bhh0 = gru_bhh0.reshape(1, 3 * H)
    bih1 = gru_bih1.reshape(1, 3 * H)
    bhh1 = gru_bhh1.reshape(1, 3 * H)
    linWT = jnp.zeros((H, H), jnp.float32).at[:, :2].set(lin_W.T)
    linb = jnp.zeros((1, H), jnp.float32).at[:, :2].set(lin_b.reshape(1, 2))

    cnt = _deg_kernel(dst)
    scaled1, dinv = _prep_call(cnt[0], cnt[1], x_pad, conv_W0)
    agg1 = _agg_kernel(scaled1, src, dst)
    (scaled2,) = _mid_call(agg1[0], agg1[1], scaled1, dinv, b0, wih0T, bih0,
                           bhh0, conv_W1)
    agg2 = _agg_kernel(scaled2, src, dst)
    (res,) = _fin_call(agg2[0], agg2[1], scaled2, dinv, b1, wih1T, bih1,
                       bhh1, linWT, linb)
    return res[:N, :2]


# speedup vs baseline: 9.7443x; 1.1230x over previous
import functools

import jax
import jax.numpy as jnp
from jax import lax
from jax.experimental import pallas as pl
from jax.experimental.pallas import tpu as pltpu
from jax.experimental.pallas import tpu_sc as plsc

N = 10000
E = 320000
H = 128

NPAD = 10240
NCORES = 2
NSUB = 16
NTILES = NCORES * NSUB
CHUNK = 128
EPT = 10240
NCHUNK = EPT // CHUNK
EPAD = NTILES * EPT
RPT = NPAD // NSUB
RB = 1024
PK = 16384

_mesh = plsc.VectorSubcoreMesh(core_axis_name="c", subcore_axis_name="s")


def _unpack(pk_v, jj, sr, dr, slot):
    for k in range(CHUNK // 16):
        v = pk_v[jj, pl.ds(k * 16, 16)]
        sr[slot, pl.ds(k * 16, 16)] = lax.shift_right_logical(v, 14)
        dr[slot, pl.ds(k * 16, 16)] = lax.bitwise_and(v, PK - 1)


@functools.partial(
    pl.kernel,
    out_type=jax.ShapeDtypeStruct((NCORES, NPAD, 16), jnp.float32),
    mesh=_mesh,
    scratch_types=[
        pltpu.VMEM((NCHUNK, CHUNK), jnp.int32),
        pltpu.VMEM((CHUNK, 16), jnp.float32),
        pltpu.VMEM((CHUNK, 16), jnp.float32),
        pltpu.VMEM_SHARED((NPAD, 16), jnp.float32),
    ],
)
def _deg_kernel(dsti_hbm, out_hbm, dst_v, zbuf, obuf, acc):
    c = lax.axis_index("c")
    s = lax.axis_index("s")
    t = c * NSUB + s
    pltpu.sync_copy(dsti_hbm.at[t], dst_v)

    def _fill(i, _):
        zbuf[i] = jnp.zeros((16,), jnp.float32)
        obuf[i] = jnp.ones((16,), jnp.float32)
        return 0

    lax.fori_loop(0, CHUNK, _fill, 0)

    base = s * RPT
    for k in range(RPT // CHUNK):
        pltpu.sync_copy(zbuf, acc.at[pl.ds(base + k * CHUNK, CHUNK), :])
    plsc.subcore_barrier()

    def _body(j, _):
        pltpu.sync_copy(obuf, acc.at[dst_v.at[j]], add=True)
        return 0

    lax.fori_loop(0, NCHUNK, _body, 0)
    plsc.subcore_barrier()

    for k in range(RPT // CHUNK):
        pltpu.sync_copy(acc.at[pl.ds(base + k * CHUNK, CHUNK), :], zbuf)
        pltpu.sync_copy(zbuf, out_hbm.at[c, pl.ds(base + k * CHUNK, CHUNK), :])


NBUF = 2


@functools.partial(
    pl.kernel,
    out_type=jax.ShapeDtypeStruct((NCORES, NPAD, H), jnp.float32),
    mesh=_mesh,
    scratch_types=[
        pltpu.VMEM((NCHUNK, CHUNK), jnp.int32),
        pltpu.VMEM((NBUF, CHUNK), jnp.int32),
        pltpu.VMEM((NBUF, CHUNK), jnp.int32),
        pltpu.VMEM((CHUNK, H), jnp.float32),
        pltpu.VMEM((CHUNK, H), jnp.float32),
        pltpu.VMEM_SHARED((NPAD, H), jnp.float32),
        pltpu.SemaphoreType.DMA,
        pltpu.SemaphoreType.DMA,
    ],
)
def _agg_kernel(table_hbm, pki_hbm, out_hbm, pk_v, sr, dr,
                buf0, buf1, acc, sem0, sem1):
    bufs = (buf0, buf1)
    sems = (sem0, sem1)
    c = lax.axis_index("c")
    s = lax.axis_index("s")
    t = c * NSUB + s
    pltpu.sync_copy(pki_hbm.at[t], pk_v)

    def _zero(i, _):
        for cc in range(H // 16):
            bufs[0][i, pl.ds(cc * 16, 16)] = jnp.zeros((16,), jnp.float32)
        return 0

    lax.fori_loop(0, CHUNK, _zero, 0)

    base = s * RPT
    for k in range(RPT // CHUNK):
        pltpu.sync_copy(bufs[0], acc.at[pl.ds(base + k * CHUNK, CHUNK), :])
    plsc.subcore_barrier()

    for b in range(NBUF):
        _unpack(pk_v, b, sr, dr, b)
        pltpu.async_copy(table_hbm.at[sr.at[b]], bufs[b], sems[b])

    def _body(j0, _):
        for b in range(NBUF):
            jj = NBUF * j0 + b
            pltpu.make_async_copy(table_hbm.at[sr.at[b]], bufs[b], sems[b]).wait()
            pltpu.sync_copy(bufs[b], acc.at[dr.at[b]], add=True)
            nxt = lax.rem(jj + NBUF, NCHUNK)
            _unpack(pk_v, nxt, sr, dr, b)
            pltpu.async_copy(table_hbm.at[sr.at[b]], bufs[b], sems[b])
        return 0

    lax.fori_loop(0, NCHUNK // NBUF, _body, 0)
    for b in range(NBUF):
        pltpu.make_async_copy(table_hbm.at[sr.at[b]], bufs[b], sems[b]).wait()
    plsc.subcore_barrier()

    for k in range(RPT // CHUNK):
        pltpu.sync_copy(acc.at[pl.ds(base + k * CHUNK, CHUNK), :], bufs[0])
        pltpu.sync_copy(bufs[0], out_hbm.at[c, pl.ds(base + k * CHUNK, CHUNK), :])


def _prep_body(cnt0, cnt1, x, w0, o_scaled, o_dinv):
    deg = cnt0[:, 0:1] + cnt1[:, 0:1] + 1.0
    dinv = lax.rsqrt(deg)
    hw = jnp.dot(x[:], w0[:], preferred_element_type=jnp.float32)
    o_scaled[:] = hw * dinv
    o_dinv[:] = jnp.broadcast_to(dinv, (RB, H))


def _gru(gx, bhh):
    r = jax.nn.sigmoid(gx[:, 0:H] + bhh[:, 0:H])
    z = jax.nn.sigmoid(gx[:, H:2 * H] + bhh[:, H:2 * H])
    n = jnp.tanh(gx[:, 2 * H:3 * H] + r * bhh[:, 2 * H:3 * H])
    return (1.0 - z) * n


def _mid_body(agg0, agg1, scaled, dinv, b, wihT, bih, bhh, w_next, o_scaled2):
    conv = dinv[:] * (agg0[:] + agg1[:] + scaled[:]) + b[:]
    a = jnp.maximum(conv, 0.0)
    gx = jnp.dot(a, wihT[:], preferred_element_type=jnp.float32) + bih[:]
    h1 = _gru(gx, bhh[:])
    hw2 = jnp.dot(h1, w_next[:], preferred_element_type=jnp.float32)
    o_scaled2[:] = hw2 * dinv[:]


def _fin_body(agg0, agg1, scaled, dinv, b, wihT, bih, bhh, linWT, linb, o):
    conv = dinv[:] * (agg0[:] + agg1[:] + scaled[:]) + b[:]
    a = jnp.maximum(conv, 0.0)
    gx = jnp.dot(a, wihT[:], preferred_element_type=jnp.float32) + bih[:]
    h2 = _gru(gx, bhh[:])
    o[:] = jnp.dot(h2, linWT[:], preferred_element_type=jnp.float32) + linb[:]


_row = pl.BlockSpec((RB, H), lambda i: (i, 0))
_row16 = pl.BlockSpec((RB, 16), lambda i: (i, 0))
_w128 = pl.BlockSpec((H, H), lambda i: (0, 0))
_w384 = pl.BlockSpec((H, 3 * H), lambda i: (0, 0))
_b128 = pl.BlockSpec((1, H), lambda i: (0, 0))
_b384 = pl.BlockSpec((1, 3 * H), lambda i: (0, 0))
_GRID = (NPAD // RB,)

_prep_call = pl.pallas_call(
    _prep_body,
    grid=_GRID,
    in_specs=[_row16, _row16, _row, _w128],
    out_specs=[_row, _row],
    out_shape=[
        jax.ShapeDtypeStruct((NPAD, H), jnp.float32),
        jax.ShapeDtypeStruct((NPAD, H), jnp.float32),
    ],
)

_mid_call = pl.pallas_call(
    _mid_body,
    grid=_GRID,
    in_specs=[_row, _row, _row, _row, _b128, _w384, _b384, _b384, _w128],
    out_specs=[_row],
    out_shape=[jax.ShapeDtypeStruct((NPAD, H), jnp.float32)],
)

_fin_call = pl.pallas_call(
    _fin_body,
    grid=_GRID,
    in_specs=[_row, _row, _row, _row, _b128, _w384, _b384, _b384, _w128, _b128],
    out_specs=[_row],
    out_shape=[jax.ShapeDtypeStruct((NPAD, H), jnp.float32)],
)


@jax.jit
def kernel(x, edge_index, conv_W0, conv_b0, conv_W1, conv_b1, gru_Wih0,
           gru_Whh0, gru_bih0, gru_bhh0, gru_Wih1, gru_Whh1, gru_bih1,
           gru_bhh1, lin_W, lin_b):
    x_pad = jnp.zeros((NPAD, H), jnp.float32).at[:N].set(x)
    pki = jnp.concatenate(
        [edge_index[0] * PK + edge_index[1],
         jnp.full((EPAD - E,), N, jnp.int32)]).reshape(NTILES, NCHUNK, CHUNK)
    dst3 = jnp.concatenate(
        [edge_index[1], jnp.full((EPAD - E,), N, jnp.int32)]).reshape(
            NTILES, NCHUNK, CHUNK)
    b0 = conv_b0.reshape(1, H)
    b1 = conv_b1.reshape(1, H)
    wih0T = gru_Wih0.T
    wih1T = gru_Wih1.T
    bih0 = gru_bih0.reshape(1, 3 * H)
    bhh0 = gru_bhh0.reshape(1, 3 * H)
    bih1 = gru_bih1.reshape(1, 3 * H)
    bhh1 = gru_bhh1.reshape(1, 3 * H)
    linWT = jnp.zeros((H, H), jnp.float32).at[:, :2].set(lin_W.T)
    linb = jnp.zeros((1, H), jnp.float32).at[:, :2].set(lin_b.reshape(1, 2))

    cnt = _deg_kernel(dst3)
    scaled1, dinv = _prep_call(cnt[0], cnt[1], x_pad, conv_W0)
    agg1 = _agg_kernel(scaled1, pki)
    (scaled2,) = _mid_call(agg1[0], agg1[1], scaled1, dinv, b0, wih0T, bih0,
                           bhh0, conv_W1)
    agg2 = _agg_kernel(scaled2, pki)
    (res,) = _fin_call(agg2[0], agg2[1], scaled2, dinv, b1, wih1T, bih1,
                       bhh1, linWT, linb)
    return res[:N, :2]
